# Initial kernel scaffold; baseline (speedup 1.0000x reference)
import jax
import jax.numpy as jnp
from jax.experimental import pallas as pl


def kernel(input_ids, weight):
    raise NotImplementedError("write your pallas kernel here")


# speedup vs baseline: 7.9195x; 7.9195x over previous
import functools

import jax
import jax.numpy as jnp
from jax import lax
from jax.experimental import pallas as pl
from jax.experimental.pallas import tpu as pltpu
from jax.experimental.pallas import tpu_sc as plsc

_NC = 2
_NS = 16
_NW = _NC * _NS
_C = 128
_NBUF = 2


@functools.lru_cache(maxsize=None)
def _make_lookup(b_total: int, d: int):
    bpw = b_total // _NW
    nchunk = bpw // _C
    mesh = plsc.VectorSubcoreMesh(
        core_axis_name="c", subcore_axis_name="s",
        num_cores=_NC, num_subcores=_NS,
    )

    @functools.partial(
        pl.kernel,
        out_type=jax.ShapeDtypeStruct((b_total, d), jnp.float32),
        mesh=mesh,
        scratch_types=[
            pltpu.VMEM((nchunk, _C), jnp.int32),
            pltpu.VMEM((_C, d), jnp.float32),
            pltpu.VMEM((_C, d), jnp.float32),
            pltpu.SemaphoreType.DMA,
            pltpu.SemaphoreType.DMA,
        ],
    )
    def lookup(table_hbm, idx_hbm, out_hbm, idx_v, buf0, buf1, sem0, sem1):
        wid = lax.axis_index("s") * _NC + lax.axis_index("c")
        base = wid * bpw
        pltpu.sync_copy(idx_hbm.at[wid], idx_v)

        bufs = (buf0, buf1)
        sems = (sem0, sem1)
        for b in range(_NBUF):
            pltpu.async_copy(table_hbm.at[idx_v.at[b]], bufs[b], sems[b])

        @pl.loop(0, nchunk, step=_NBUF)
        def _(g):
            for b in range(_NBUF):
                c = g + b
                pltpu.make_async_copy(
                    table_hbm.at[idx_v.at[c]], bufs[b], sems[b]).wait()
                pltpu.sync_copy(bufs[b], out_hbm.at[pl.ds(base + c * _C, _C)])
                nxt = c + _NBUF

                @pl.when(nxt < nchunk)
                def _():
                    pltpu.async_copy(
                        table_hbm.at[idx_v.at[nxt]], bufs[b], sems[b])

    return lookup


def kernel(input_ids, weight):
    orig_shape = input_ids.shape
    d = weight.shape[1]
    flat = jnp.reshape(input_ids, (-1,)).astype(jnp.int32)
    b = flat.shape[0]
    blk = _NW * _C
    b_pad = ((b + blk - 1) // blk) * blk
    if b_pad != b:
        flat = jnp.concatenate(
            [flat, jnp.zeros((b_pad - b,), jnp.int32)])
    idx = flat.reshape(_NW, b_pad // (_NW * _C), _C)
    out = _make_lookup(b_pad, d)(weight.astype(jnp.float32), idx)
    if b_pad != b:
        out = out[:b]
    return jnp.reshape(out, orig_shape + (d,))


# speedup vs baseline: 8.0607x; 1.0178x over previous
import functools

import jax
import jax.numpy as jnp
from jax import lax
from jax.experimental import pallas as pl
from jax.experimental.pallas import tpu as pltpu
from jax.experimental.pallas import tpu_sc as plsc

_NC = 2
_NS = 16
_NW = _NC * _NS
_C = 128
_NBUF = 5


@functools.lru_cache(maxsize=None)
def _make_lookup(b_total: int, d: int):
    bpw = b_total // _NW
    nchunk = bpw // _C
    mesh = plsc.VectorSubcoreMesh(
        core_axis_name="c", subcore_axis_name="s",
        num_cores=_NC, num_subcores=_NS,
    )

    @functools.partial(
        pl.kernel,
        out_type=jax.ShapeDtypeStruct((b_total, d), jnp.float32),
        mesh=mesh,
        scratch_types=[
            pltpu.VMEM((nchunk, _C), jnp.int32),
        ] + [pltpu.VMEM((_C, d), jnp.float32)] * _NBUF
          + [pltpu.SemaphoreType.DMA] * _NBUF,
    )
    def lookup(table_hbm, idx_hbm, out_hbm, idx_v, *bufs_and_sems):
        bufs = bufs_and_sems[:_NBUF]
        sems = bufs_and_sems[_NBUF:]
        wid = lax.axis_index("s") * _NC + lax.axis_index("c")
        base = wid * bpw
        pltpu.sync_copy(idx_hbm.at[wid], idx_v)

        for b in range(_NBUF):
            pltpu.async_copy(table_hbm.at[idx_v.at[b]], bufs[b], sems[b])

        @pl.loop(0, nchunk, step=_NBUF)
        def _(g):
            for b in range(_NBUF):
                c = g + b
                pltpu.make_async_copy(
                    table_hbm.at[idx_v.at[c]], bufs[b], sems[b]).wait()
                pltpu.sync_copy(bufs[b], out_hbm.at[pl.ds(base + c * _C, _C)])
                nxt = c + _NBUF

                @pl.when(nxt < nchunk)
                def _():
                    pltpu.async_copy(
                        table_hbm.at[idx_v.at[nxt]], bufs[b], sems[b])

    return lookup


def kernel(input_ids, weight):
    orig_shape = input_ids.shape
    d = weight.shape[1]
    flat = jnp.reshape(input_ids, (-1,)).astype(jnp.int32)
    b = flat.shape[0]
    blk = _NW * _C
    b_pad = ((b + blk - 1) // blk) * blk
    if b_pad != b:
        flat = jnp.concatenate(
            [flat, jnp.zeros((b_pad - b,), jnp.int32)])
    idx = flat.reshape(_NW, b_pad // (_NW * _C), _C)
    out = _make_lookup(b_pad, d)(weight.astype(jnp.float32), idx)
    if b_pad != b:
        out = out[:b]
    return jnp.reshape(out, orig_shape + (d,))
